# baseline (speedup 1.0000x reference)
import jax
import jax.numpy as jnp
from jax import lax
from jax.experimental import pallas as pl
from jax.experimental.pallas import tpu as pltpu
from jax.experimental.pallas import tpu_sc as plsc

TF = 1600
DEC = 12800
GENES = 18000
B = 1024

BLK = 32
NBLK = B // BLK
NSC = 2
NTEC = 16
TILE = 128
RING = 2

NT1 = 14
NT2 = 26
NT3 = 72

GENES_PAD = 18048
H2_OFF = 0
H1_OFF = DEC
OS_OFF = DEC
SPM_ROWS = DEC + GENES_PAD

H_CH = DEC // NTEC
O_CH = GENES_PAD // NTEC


def _pad_edges(rows, cols, w, n_pad):
    n = rows.shape[0]
    pad = n_pad - n
    rows = jnp.concatenate([rows.astype(jnp.int32), jnp.zeros((pad,), jnp.int32)])
    cols = jnp.concatenate([cols.astype(jnp.int32), jnp.zeros((pad,), jnp.int32)])
    w = jnp.concatenate([w, jnp.zeros((pad,), w.dtype)])
    return rows, cols, w


def _sc_body(x_ref, c1_ref, r1_ref, w1_ref, b1_ref,
             c2_ref, r2_ref, w2_ref, b2_ref,
             c3_ref, r3_ref, w3_ref, b3_ref,
             out_ref,
             spm, colsv1b, colsv1, rowsv1, colsv2, rowsv2,
             colsv3, rowsv3, gbuf, cbuf, wbuf,
             gsem, wsem, ssem, isem):
    c = lax.axis_index("c")
    t = lax.axis_index("s")

    pltpu.sync_copy(c1_ref.at[t], colsv1b)
    pltpu.sync_copy(r1_ref.at[t], rowsv1)
    pltpu.sync_copy(c2_ref.at[t], colsv2)
    pltpu.sync_copy(r2_ref.at[t], rowsv2)
    pltpu.sync_copy(c3_ref.at[t], colsv3)
    pltpu.sync_copy(r3_ref.at[t], rowsv3)

    def run_layer(src, colsv, rowsv, w_hbm, nt, relu):
        def g_start(tile, s):
            pltpu.async_copy(src.at[colsv.at[tile]], gbuf.at[s], gsem.at[s])
            pltpu.async_copy(w_hbm.at[t, tile], wbuf.at[s], wsem.at[s])

        def g_wait(tile, s):
            pltpu.make_async_copy(src.at[colsv.at[tile]], gbuf.at[s],
                                  gsem.at[s]).wait()
            pltpu.make_async_copy(w_hbm.at[t, tile], wbuf.at[s],
                                  wsem.at[s]).wait()

        def s_start(tile, sc):
            pltpu.async_copy(cbuf.at[sc], spm.at[rowsv.at[tile]], ssem.at[sc],
                             add=True)

        def s_wait(tile, sc):
            pltpu.make_async_copy(cbuf.at[sc], spm.at[rowsv.at[tile]],
                                  ssem.at[sc]).wait()

        def compute(s, sc):
            gb = gbuf.at[s]
            wb = wbuf.at[s]
            cb = cbuf.at[sc]

            @plsc.parallel_loop(0, TILE, unroll=4)
            def _(e):
                wspl = wb[e, :]
                for v in range(BLK // 16):
                    x = gb[e, pl.ds(16 * v, 16)]
                    if relu:
                        x = jnp.maximum(x, x * 0.01)
                    cb[e, pl.ds(16 * v, 16)] = x * wspl

        for s in range(RING):
            g_start(s, s)

        @pl.loop(0, nt // RING)
        def _(it):
            tt = RING * it
            for s in range(RING):
                sc = s % 2
                g_wait(tt + s, s)

                @pl.when(tt + s >= 2)
                def _():
                    s_wait(tt + s - 2, sc)

                compute(s, sc)
                s_start(tt + s, sc)

                @pl.when(tt + s + RING < nt)
                def _():
                    g_start(tt + s + RING, s)

        s_wait(nt - 2, 0)
        s_wait(nt - 1, 1)

    bps = NBLK // NSC

    @pl.loop(c * bps, (c + 1) * bps)
    def _(j):
        i1 = pltpu.async_copy(b1_ref.at[pl.ds(t * H_CH, H_CH)],
                              spm.at[pl.ds(H1_OFF + t * H_CH, H_CH)],
                              isem.at[0])
        i2 = pltpu.async_copy(b2_ref.at[pl.ds(t * H_CH, H_CH)],
                              spm.at[pl.ds(H2_OFF + t * H_CH, H_CH)],
                              isem.at[1])

        off16 = jnp.full((16,), j * TF, jnp.int32)

        @pl.loop(0, NT1)
        def _(tile):
            for k in range(TILE // 16):
                colsv1[tile, pl.ds(16 * k, 16)] = (
                    colsv1b[tile, pl.ds(16 * k, 16)] + off16)

        i1.wait()
        i2.wait()
        plsc.subcore_barrier()

        run_layer(x_ref, colsv1, rowsv1, w1_ref, NT1, relu=False)
        plsc.subcore_barrier()

        run_layer(spm, colsv2, rowsv2, w2_ref, NT2, relu=True)
        plsc.subcore_barrier()

        pltpu.sync_copy(b3_ref.at[pl.ds(t * O_CH, O_CH)],
                        spm.at[pl.ds(OS_OFF + t * O_CH, O_CH)])
        plsc.subcore_barrier()

        run_layer(spm, colsv3, rowsv3, w3_ref, NT3, relu=True)
        plsc.subcore_barrier()

        pltpu.sync_copy(spm.at[pl.ds(OS_OFF + t * O_CH, O_CH)],
                        out_ref.at[j, pl.ds(t * O_CH, O_CH)])
        plsc.subcore_barrier()


@jax.jit
def _decoder(features, rows1, cols1, w1, b1, rows2, cols2, w2, b2,
             rows3, cols3, w3, b3):
    f32 = jnp.float32

    one = (w1[0] * 0.0 + 1.0).astype(f32)

    x_b = (features * one).reshape(NBLK, BLK, TF).transpose(0, 2, 1).reshape(NBLK * TF, BLK)

    r1, c1, ww1 = _pad_edges(rows1, cols1, w1, NT1 * TILE * NTEC)
    r2, c2, ww2 = _pad_edges(rows2, cols2, w2, NT2 * TILE * NTEC)
    r3, c3, ww3 = _pad_edges(rows3, cols3, w3, NT3 * TILE * NTEC)

    c1_b = c1.reshape(NTEC, NT1, TILE)
    r1_b = (r1 + H1_OFF).reshape(NTEC, NT1, TILE)
    w1_b = jnp.broadcast_to((ww1 * one)[:, None], (NT1 * TILE * NTEC, 16)).reshape(NTEC, NT1, TILE, 16)
    c2_b = (c2 + H1_OFF).reshape(NTEC, NT2, TILE)
    r2_b = (r2 + H2_OFF).reshape(NTEC, NT2, TILE)
    w2_b = jnp.broadcast_to((ww2 * one)[:, None], (NT2 * TILE * NTEC, 16)).reshape(NTEC, NT2, TILE, 16)
    c3_b = (c3 + H2_OFF).reshape(NTEC, NT3, TILE)
    r3_b = (r3 + OS_OFF).reshape(NTEC, NT3, TILE)
    w3_b = jnp.broadcast_to((ww3 * one)[:, None], (NT3 * TILE * NTEC, 16)).reshape(NTEC, NT3, TILE, 16)

    b1_bc = jnp.broadcast_to(b1[:, None], (DEC, BLK)).astype(f32)
    b2_bc = jnp.broadcast_to(b2[:, None], (DEC, BLK)).astype(f32)
    b3_bc = jnp.broadcast_to(b3[:, None], (GENES, BLK)).astype(f32)
    b3_bc = jnp.concatenate([b3_bc, jnp.zeros((GENES_PAD - GENES, BLK), f32)])

    sc_call = pl.kernel(
        _sc_body,
        out_type=jax.ShapeDtypeStruct((NBLK, GENES_PAD, BLK), f32),
        mesh=plsc.VectorSubcoreMesh(core_axis_name="c", subcore_axis_name="s"),
        compiler_params=pltpu.CompilerParams(use_tc_tiling_on_sc=False),
        scratch_types=[
            pltpu.VMEM_SHARED((SPM_ROWS, BLK), f32),
            pltpu.VMEM((NT1, TILE), jnp.int32),
            pltpu.VMEM((NT1, TILE), jnp.int32),
            pltpu.VMEM((NT1, TILE), jnp.int32),
            pltpu.VMEM((NT2, TILE), jnp.int32),
            pltpu.VMEM((NT2, TILE), jnp.int32),
            pltpu.VMEM((NT3, TILE), jnp.int32),
            pltpu.VMEM((NT3, TILE), jnp.int32),
            pltpu.VMEM((RING, TILE, BLK), f32),
            pltpu.VMEM((2, TILE, BLK), f32),
            pltpu.VMEM((RING, TILE, 16), f32),
            pltpu.SemaphoreType.DMA((RING,)),
            pltpu.SemaphoreType.DMA((RING,)),
            pltpu.SemaphoreType.DMA((2,)),
            pltpu.SemaphoreType.DMA((2,)),
        ],
    )
    out_b = sc_call(x_b, c1_b, r1_b, w1_b, b1_bc,
                    c2_b, r2_b, w2_b, b2_bc,
                    c3_b, r3_b, w3_b, b3_bc)
    return (out_b.transpose(0, 2, 1).reshape(B, GENES_PAD)[:, :GENES]) * one


def kernel(features, rows1, cols1, w1, b1, rows2, cols2, w2, b2,
           rows3, cols3, w3, b3):
    return _decoder(features, rows1, cols1, w1, b1, rows2, cols2, w2, b2,
                    rows3, cols3, w3, b3)


# speedup vs baseline: 1.3586x; 1.3586x over previous
import jax
import jax.numpy as jnp
from jax import lax
from jax.experimental import pallas as pl
from jax.experimental.pallas import tpu as pltpu
from jax.experimental.pallas import tpu_sc as plsc

TF = 1600
DEC = 12800
GENES = 18000
B = 1024

BLK = 32
NBLK = B // BLK
NSC = 2
NTEC = 16
TILE = 128
RING = 2

NT1 = 14
NT2 = 26
NT3 = 72

GENES_PAD = 18048
H2_OFF = 0
H1_OFF = DEC
OS_OFF = DEC
SPM_ROWS = DEC + GENES_PAD

H_CH = DEC // NTEC
O_CH = GENES_PAD // NTEC


def _pad_edges(rows, cols, w, n_pad):
    n = rows.shape[0]
    pad = n_pad - n
    rows = jnp.concatenate([rows.astype(jnp.int32), jnp.zeros((pad,), jnp.int32)])
    cols = jnp.concatenate([cols.astype(jnp.int32), jnp.zeros((pad,), jnp.int32)])
    w = jnp.concatenate([w, jnp.zeros((pad,), w.dtype)])
    return rows, cols, w


def _sc_body(x_ref, c1_ref, r1_ref, w1_ref, b1_ref,
             c2_ref, r2_ref, w2_ref, b2_ref,
             c3_ref, r3_ref, w3_ref, b3_ref,
             out_ref,
             spm, xs, colsv1, rowsv1, colsv2, rowsv2,
             colsv3, rowsv3, gbuf, cbuf, wbuf,
             gsem, wsem, ssem, isem):
    c = lax.axis_index("c")
    t = lax.axis_index("s")

    pltpu.sync_copy(c1_ref.at[t], colsv1)
    pltpu.sync_copy(r1_ref.at[t], rowsv1)
    pltpu.sync_copy(c2_ref.at[t], colsv2)
    pltpu.sync_copy(r2_ref.at[t], rowsv2)
    pltpu.sync_copy(c3_ref.at[t], colsv3)
    pltpu.sync_copy(r3_ref.at[t], rowsv3)

    def run_layer(src, colsv, rowsv, w_hbm, nt, relu):
        def g_start(tile, s):
            pltpu.async_copy(src.at[colsv.at[tile]], gbuf.at[s], gsem.at[s])
            pltpu.async_copy(w_hbm.at[t, tile], wbuf.at[s], wsem.at[s])

        def g_wait(tile, s):
            pltpu.make_async_copy(src.at[colsv.at[tile]], gbuf.at[s],
                                  gsem.at[s]).wait()
            pltpu.make_async_copy(w_hbm.at[t, tile], wbuf.at[s],
                                  wsem.at[s]).wait()

        def s_start(tile, sc):
            pltpu.async_copy(cbuf.at[sc], spm.at[rowsv.at[tile]], ssem.at[sc],
                             add=True)

        def s_wait(tile, sc):
            pltpu.make_async_copy(cbuf.at[sc], spm.at[rowsv.at[tile]],
                                  ssem.at[sc]).wait()

        def compute(s, sc):
            gb = gbuf.at[s]
            wb = wbuf.at[s]
            cb = cbuf.at[sc]

            @plsc.parallel_loop(0, TILE, unroll=4)
            def _(e):
                wspl = wb[e, :]
                for v in range(BLK // 16):
                    x = gb[e, pl.ds(16 * v, 16)]
                    if relu:
                        x = jnp.maximum(x, x * 0.01)
                    cb[e, pl.ds(16 * v, 16)] = x * wspl

        for s in range(RING):
            g_start(s, s)

        @pl.loop(0, nt // RING)
        def _(it):
            tt = RING * it
            for s in range(RING):
                sc = s % 2
                g_wait(tt + s, s)

                @pl.when(tt + s >= 2)
                def _():
                    s_wait(tt + s - 2, sc)

                compute(s, sc)
                s_start(tt + s, sc)

                @pl.when(tt + s + RING < nt)
                def _():
                    g_start(tt + s + RING, s)

        s_wait(nt - 2, 0)
        s_wait(nt - 1, 1)

    bps = NBLK // NSC

    @pl.loop(c * bps, (c + 1) * bps)
    def _(j):
        i1 = pltpu.async_copy(b1_ref.at[pl.ds(t * H_CH, H_CH)],
                              spm.at[pl.ds(H1_OFF + t * H_CH, H_CH)],
                              isem.at[0])
        i2 = pltpu.async_copy(b2_ref.at[pl.ds(t * H_CH, H_CH)],
                              spm.at[pl.ds(H2_OFF + t * H_CH, H_CH)],
                              isem.at[1])

        pltpu.sync_copy(x_ref.at[pl.ds(j * TF + t * (TF // NTEC), TF // NTEC)],
                        xs.at[pl.ds(t * (TF // NTEC), TF // NTEC)])

        i1.wait()
        i2.wait()
        plsc.subcore_barrier()

        run_layer(xs, colsv1, rowsv1, w1_ref, NT1, relu=False)
        plsc.subcore_barrier()

        run_layer(spm, colsv2, rowsv2, w2_ref, NT2, relu=True)
        plsc.subcore_barrier()

        pltpu.sync_copy(b3_ref.at[pl.ds(t * O_CH, O_CH)],
                        spm.at[pl.ds(OS_OFF + t * O_CH, O_CH)])
        plsc.subcore_barrier()

        run_layer(spm, colsv3, rowsv3, w3_ref, NT3, relu=True)
        plsc.subcore_barrier()

        pltpu.sync_copy(spm.at[pl.ds(OS_OFF + t * O_CH, O_CH)],
                        out_ref.at[j, pl.ds(t * O_CH, O_CH)])
        plsc.subcore_barrier()


@jax.jit
def _decoder(features, rows1, cols1, w1, b1, rows2, cols2, w2, b2,
             rows3, cols3, w3, b3):
    f32 = jnp.float32

    one = (w1[0] * 0.0 + 1.0).astype(f32)

    x_b = (features * one).reshape(NBLK, BLK, TF).transpose(0, 2, 1).reshape(NBLK * TF, BLK)

    r1, c1, ww1 = _pad_edges(rows1, cols1, w1, NT1 * TILE * NTEC)
    r2, c2, ww2 = _pad_edges(rows2, cols2, w2, NT2 * TILE * NTEC)
    r3, c3, ww3 = _pad_edges(rows3, cols3, w3, NT3 * TILE * NTEC)

    c1_b = c1.reshape(NTEC, NT1, TILE)
    r1_b = (r1 + H1_OFF).reshape(NTEC, NT1, TILE)
    w1_b = jnp.broadcast_to((ww1 * one)[:, None], (NT1 * TILE * NTEC, 16)).reshape(NTEC, NT1, TILE, 16)
    c2_b = (c2 + H1_OFF).reshape(NTEC, NT2, TILE)
    r2_b = (r2 + H2_OFF).reshape(NTEC, NT2, TILE)
    w2_b = jnp.broadcast_to((ww2 * one)[:, None], (NT2 * TILE * NTEC, 16)).reshape(NTEC, NT2, TILE, 16)
    c3_b = (c3 + H2_OFF).reshape(NTEC, NT3, TILE)
    r3_b = (r3 + OS_OFF).reshape(NTEC, NT3, TILE)
    w3_b = jnp.broadcast_to((ww3 * one)[:, None], (NT3 * TILE * NTEC, 16)).reshape(NTEC, NT3, TILE, 16)

    b1_bc = jnp.broadcast_to(b1[:, None], (DEC, BLK)).astype(f32)
    b2_bc = jnp.broadcast_to(b2[:, None], (DEC, BLK)).astype(f32)
    b3_bc = jnp.broadcast_to(b3[:, None], (GENES, BLK)).astype(f32)
    b3_bc = jnp.concatenate([b3_bc, jnp.zeros((GENES_PAD - GENES, BLK), f32)])

    sc_call = pl.kernel(
        _sc_body,
        out_type=jax.ShapeDtypeStruct((NBLK, GENES_PAD, BLK), f32),
        mesh=plsc.VectorSubcoreMesh(core_axis_name="c", subcore_axis_name="s"),
        compiler_params=pltpu.CompilerParams(use_tc_tiling_on_sc=False),
        scratch_types=[
            pltpu.VMEM_SHARED((SPM_ROWS, BLK), f32),
            pltpu.VMEM_SHARED((TF, BLK), f32),
            pltpu.VMEM((NT1, TILE), jnp.int32),
            pltpu.VMEM((NT1, TILE), jnp.int32),
            pltpu.VMEM((NT2, TILE), jnp.int32),
            pltpu.VMEM((NT2, TILE), jnp.int32),
            pltpu.VMEM((NT3, TILE), jnp.int32),
            pltpu.VMEM((NT3, TILE), jnp.int32),
            pltpu.VMEM((RING, TILE, BLK), f32),
            pltpu.VMEM((2, TILE, BLK), f32),
            pltpu.VMEM((RING, TILE, 16), f32),
            pltpu.SemaphoreType.DMA((RING,)),
            pltpu.SemaphoreType.DMA((RING,)),
            pltpu.SemaphoreType.DMA((2,)),
            pltpu.SemaphoreType.DMA((2,)),
        ],
    )
    out_b = sc_call(x_b, c1_b, r1_b, w1_b, b1_bc,
                    c2_b, r2_b, w2_b, b2_bc,
                    c3_b, r3_b, w3_b, b3_bc)
    return (out_b.transpose(0, 2, 1).reshape(B, GENES_PAD)[:, :GENES]) * one


def kernel(features, rows1, cols1, w1, b1, rows2, cols2, w2, b2,
           rows3, cols3, w3, b3):
    return _decoder(features, rows1, cols1, w1, b1, rows2, cols2, w2, b2,
                    rows3, cols3, w3, b3)
